# baseline (speedup 1.0000x reference)
import functools

import jax
import jax.numpy as jnp
from jax import lax
from jax.experimental import pallas as pl
from jax.experimental.pallas import tpu as pltpu
from jax.experimental.pallas import tpu_sc as plsc

_N = 10000
_E = 320000
_H = 128
_EPS = 1e-5

_NC = 2
_NS = 16
_L = 16
_NW = _NC * _NS
_EPT = _E // _NW
_CHUNKS = _EPT // _L
_UNROLL = 25


def _tc_logits_body(x_ref, w_ref, wa2_ref, b_ref, pq_ref):
    wa2 = wa2_ref[...]
    uv = jax.lax.dot_general(wa2, w_ref[...], (((1,), (0,)), ((), ())),
                             preferred_element_type=jnp.float32)
    pq = jax.lax.dot_general(uv, x_ref[...], (((1,), (1,)), ((), ())),
                             preferred_element_type=jnp.float32)
    c = jnp.sum((wa2[0:1, :] + wa2[1:2, :]) * b_ref[...])
    row = jax.lax.broadcasted_iota(jnp.int32, (2, _N), 0)
    pq_ref[...] = pq + jnp.where(row == 1, c, jnp.float32(0.0))


def _sc_edge_body(pq_hbm, e_hbm, out_hbm, p_v, q_v, e_v, n_v):
    cid = lax.axis_index("c")
    sid = lax.axis_index("s")
    wid = sid * _NC + cid
    base = pl.multiple_of(wid * _EPT, 8)

    pltpu.sync_copy(e_hbm.at[pl.ds(base * 2, 2 * _EPT)], e_v)
    pltpu.sync_copy(pq_hbm.at[0], p_v)
    pltpu.sync_copy(pq_hbm.at[1], q_v)

    riota2 = lax.iota(jnp.int32, _L) * 2

    def _zero(i, _):
        b0 = pl.multiple_of(i * (_L * _UNROLL), _L)
        for j in range(_UNROLL):
            n_v[pl.ds(b0 + j * _L, _L)] = jnp.zeros((_L,), jnp.float32)
        return _

    lax.fori_loop(0, _CHUNKS // _UNROLL, _zero, None)

    def _step(i, _):
        b0 = pl.multiple_of(i * (_L * _UNROLL), _L)
        for j in range(_UNROLL):
            idx0 = riota2 + (2 * (b0 + j * _L))
            i0 = plsc.load_gather(e_v, [idx0])
            i1 = plsc.load_gather(e_v, [idx0 + 1])
            p = plsc.load_gather(p_v, [i0])
            q = plsc.load_gather(q_v, [i1])
            z = p + q
            a = jnp.exp(jnp.where(z > 0.0, z, 0.2 * z))
            plsc.addupdate_scatter(n_v, [i1], a)
        return _

    lax.fori_loop(0, _CHUNKS // _UNROLL, _step, None)

    pltpu.sync_copy(n_v, out_hbm.at[wid])


def _tc_norm_body(x_ref, np_ref, w_ref, b_ref, o_ref):
    x = x_ref[...]
    nsum = jnp.sum(np_ref[...], axis=0)
    n = nsum.reshape(_N, 1)
    den = jnp.where(n == 0.0, jnp.float32(1.0), n)
    h = jnp.maximum(x / den, 0.0) + x
    inv = jax.lax.rsqrt(jnp.mean(h * h, axis=1, keepdims=True) + _EPS)
    o_ref[...] = h * inv * w_ref[...] + b_ref[...]


def _edge_nsum(pq, e):
    mesh = plsc.VectorSubcoreMesh(core_axis_name="c", subcore_axis_name="s",
                                  num_cores=_NC, num_subcores=_NS)
    return pl.kernel(
        _sc_edge_body,
        out_type=jax.ShapeDtypeStruct((_NW, _N), jnp.float32),
        mesh=mesh,
        compiler_params=pltpu.CompilerParams(needs_layout_passes=False),
        scratch_types=[
            pltpu.VMEM((_N,), jnp.float32),
            pltpu.VMEM((_N,), jnp.float32),
            pltpu.VMEM((2 * _EPT,), jnp.int32),
            pltpu.VMEM((_N,), jnp.float32),
        ],
    )(pq, e)


def kernel(x, unused, e, W, b, Wa, rms_w, rms_b):
    x = x.astype(jnp.float32)
    e = e.astype(jnp.int32)

    pq = pl.pallas_call(
        _tc_logits_body,
        out_shape=jax.ShapeDtypeStruct((2, _N), jnp.float32),
    )(x, W, Wa.reshape(2, _H), b.reshape(1, _H))

    nparts = _edge_nsum(pq, e.reshape(2 * _E))

    out = pl.pallas_call(
        _tc_norm_body,
        out_shape=jax.ShapeDtypeStruct((_N, _H), jnp.float32),
    )(x, nparts, rms_w.reshape(1, _H), rms_b.reshape(1, _H))
    return out


# speedup vs baseline: 3.6071x; 3.6071x over previous
import functools

import jax
import jax.numpy as jnp
from jax import lax
from jax.experimental import pallas as pl
from jax.experimental.pallas import tpu as pltpu
from jax.experimental.pallas import tpu_sc as plsc

_N = 10000
_E = 320000
_H = 128
_EPS = 1e-5

_NC = 2
_NS = 16
_L = 16
_NW = _NC * _NS
_EPT = _E // _NW
_CHUNKS = _EPT // _L
_UNROLL = 25


def _tc_logits_body(x_ref, w_ref, wa2_ref, b_ref, pq_ref):
    wa2 = wa2_ref[...]
    uv = jax.lax.dot_general(wa2, w_ref[...], (((1,), (0,)), ((), ())),
                             preferred_element_type=jnp.float32)
    pq = jax.lax.dot_general(uv, x_ref[...], (((1,), (1,)), ((), ())),
                             preferred_element_type=jnp.float32)
    c = jnp.sum((wa2[0:1, :] + wa2[1:2, :]) * b_ref[...])
    row = jax.lax.broadcasted_iota(jnp.int32, (2, _N), 0)
    pq_ref[...] = pq + jnp.where(row == 1, c, jnp.float32(0.0))


def _sc_edge_body(pq_hbm, e_hbm, out_hbm, p_v, q_v, e_v, n_v):
    cid = lax.axis_index("c")
    sid = lax.axis_index("s")
    wid = sid * _NC + cid
    base = pl.multiple_of(wid * _EPT, 8)

    pltpu.sync_copy(e_hbm.at[pl.ds(base, _EPT)], e_v)
    pltpu.sync_copy(pq_hbm.at[0], p_v)
    pltpu.sync_copy(pq_hbm.at[1], q_v)

    def _zero(i, _):
        b0 = pl.multiple_of(i * (_L * _UNROLL), _L)
        for j in range(_UNROLL):
            n_v[pl.ds(b0 + j * _L, _L)] = jnp.zeros((_L,), jnp.float32)
        return _

    lax.fori_loop(0, _CHUNKS // _UNROLL, _zero, None)

    def _step(i, _):
        b0 = pl.multiple_of(i * (_L * _UNROLL), _L)
        for j in range(_UNROLL):
            w = e_v[pl.ds(b0 + j * _L, _L)]
            i0 = lax.shift_right_logical(w, 14)
            i1 = w & 0x3FFF
            p = plsc.load_gather(p_v, [i0])
            q = plsc.load_gather(q_v, [i1])
            z = p + q
            a = jnp.exp(jnp.where(z > 0.0, z, 0.2 * z))
            plsc.addupdate_scatter(n_v, [i1], a)
        return _

    lax.fori_loop(0, _CHUNKS // _UNROLL, _step, None)

    pltpu.sync_copy(n_v, out_hbm.at[wid])


def _tc_norm_body(x_ref, np_ref, w_ref, b_ref, o_ref):
    x = x_ref[...]
    nsum = jnp.sum(np_ref[...], axis=0)
    n = nsum.reshape(_N, 1)
    den = jnp.where(n == 0.0, jnp.float32(1.0), n)
    h = jnp.maximum(x / den, 0.0) + x
    inv = jax.lax.rsqrt(jnp.mean(h * h, axis=1, keepdims=True) + _EPS)
    o_ref[...] = h * inv * w_ref[...] + b_ref[...]


def _edge_nsum(pq, e):
    mesh = plsc.VectorSubcoreMesh(core_axis_name="c", subcore_axis_name="s",
                                  num_cores=_NC, num_subcores=_NS)
    return pl.kernel(
        _sc_edge_body,
        out_type=jax.ShapeDtypeStruct((_NW, _N), jnp.float32),
        mesh=mesh,
        compiler_params=pltpu.CompilerParams(needs_layout_passes=False),
        scratch_types=[
            pltpu.VMEM((_N,), jnp.float32),
            pltpu.VMEM((_N,), jnp.float32),
            pltpu.VMEM((_EPT,), jnp.int32),
            pltpu.VMEM((_N,), jnp.float32),
        ],
    )(pq, e)


def kernel(x, unused, e, W, b, Wa, rms_w, rms_b):
    x = x.astype(jnp.float32)
    e = e.astype(jnp.int32)

    pq = pl.pallas_call(
        _tc_logits_body,
        out_shape=jax.ShapeDtypeStruct((2, _N), jnp.float32),
    )(x, W, Wa.reshape(2, _H), b.reshape(1, _H))

    epk = (e[:, 0] << 14) | e[:, 1]
    nparts = _edge_nsum(pq, epk)

    out = pl.pallas_call(
        _tc_norm_body,
        out_shape=jax.ShapeDtypeStruct((_N, _H), jnp.float32),
    )(x, nparts, rms_w.reshape(1, _H), rms_b.reshape(1, _H))
    return out


# speedup vs baseline: 4.3399x; 1.2032x over previous
import functools

import jax
import jax.numpy as jnp
from jax import lax
from jax.experimental import pallas as pl
from jax.experimental.pallas import tpu as pltpu
from jax.experimental.pallas import tpu_sc as plsc

_N = 10000
_E = 320000
_H = 128
_EPS = 1e-5

_NC = 2
_NS = 16
_L = 16
_NW = _NC * _NS
_EPT = _E // _NW
_CHUNKS = _EPT // _L
_UNROLL = 25


def _tc_logits_body(x_ref, w_ref, wa2_ref, b_ref, pq_ref):
    wa2 = wa2_ref[...]
    uv = jax.lax.dot_general(wa2, w_ref[...], (((1,), (0,)), ((), ())),
                             preferred_element_type=jnp.float32)
    pq = jax.lax.dot_general(uv, x_ref[...], (((1,), (1,)), ((), ())),
                             preferred_element_type=jnp.float32)
    c = jnp.sum((wa2[0:1, :] + wa2[1:2, :]) * b_ref[...])
    row = jax.lax.broadcasted_iota(jnp.int32, (2, _N), 0)
    pq_ref[...] = pq + jnp.where(row == 1, c, jnp.float32(0.0))


def _sc_edge_body(pq_hbm, e_hbm, out_hbm, p_v, q_v, e_v, n_v):
    cid = lax.axis_index("c")
    sid = lax.axis_index("s")
    wid = sid * _NC + cid
    base = pl.multiple_of(wid * _EPT, 8)

    pltpu.sync_copy(e_hbm.at[pl.ds(base, _EPT)], e_v)
    pltpu.sync_copy(pq_hbm.at[0], p_v)
    pltpu.sync_copy(pq_hbm.at[1], q_v)

    def _zero(i, _):
        b0 = pl.multiple_of(i * (_L * _UNROLL), _L)
        for j in range(_UNROLL):
            n_v[pl.ds(b0 + j * _L, _L)] = jnp.zeros((_L,), jnp.float32)
        return _

    lax.fori_loop(0, _CHUNKS // _UNROLL, _zero, None)

    @plsc.parallel_loop(0, _CHUNKS, 1, unroll=_UNROLL)
    def _step(i):
        w = e_v[pl.ds(pl.multiple_of(i * _L, _L), _L)]
        i0 = lax.shift_right_logical(w, 14)
        i1 = w & 0x3FFF
        p = plsc.load_gather(p_v, [i0])
        q = plsc.load_gather(q_v, [i1])
        z = p + q
        a = jnp.exp(jnp.where(z > 0.0, z, 0.2 * z))
        plsc.addupdate_scatter(n_v, [i1], a)

    pltpu.sync_copy(n_v, out_hbm.at[wid])


def _tc_norm_body(x_ref, np_ref, w_ref, b_ref, o_ref):
    x = x_ref[...]
    nsum = jnp.sum(np_ref[...], axis=0)
    n = nsum.reshape(_N, 1)
    den = jnp.where(n == 0.0, jnp.float32(1.0), n)
    h = jnp.maximum(x / den, 0.0) + x
    inv = jax.lax.rsqrt(jnp.mean(h * h, axis=1, keepdims=True) + _EPS)
    o_ref[...] = h * inv * w_ref[...] + b_ref[...]


def _edge_nsum(pq, e):
    mesh = plsc.VectorSubcoreMesh(core_axis_name="c", subcore_axis_name="s",
                                  num_cores=_NC, num_subcores=_NS)
    return pl.kernel(
        _sc_edge_body,
        out_type=jax.ShapeDtypeStruct((_NW, _N), jnp.float32),
        mesh=mesh,
        compiler_params=pltpu.CompilerParams(needs_layout_passes=False),
        scratch_types=[
            pltpu.VMEM((_N,), jnp.float32),
            pltpu.VMEM((_N,), jnp.float32),
            pltpu.VMEM((_EPT,), jnp.int32),
            pltpu.VMEM((_N,), jnp.float32),
        ],
    )(pq, e)


def kernel(x, unused, e, W, b, Wa, rms_w, rms_b):
    x = x.astype(jnp.float32)
    e = e.astype(jnp.int32)

    pq = pl.pallas_call(
        _tc_logits_body,
        out_shape=jax.ShapeDtypeStruct((2, _N), jnp.float32),
    )(x, W, Wa.reshape(2, _H), b.reshape(1, _H))

    epk = (e[:, 0] << 14) | e[:, 1]
    nparts = _edge_nsum(pq, epk)

    out = pl.pallas_call(
        _tc_norm_body,
        out_shape=jax.ShapeDtypeStruct((_N, _H), jnp.float32),
    )(x, nparts, rms_w.reshape(1, _H), rms_b.reshape(1, _H))
    return out
